# baseline (speedup 1.0000x reference)
import jax
import jax.numpy as jnp
from jax.experimental import pallas as pl

B, V, C = 16, 50, 512
D, DFF = 256, 1024

G = 16
R = G * V


def _layer_norm(x):
    m = jnp.mean(x, axis=-1, keepdims=True)
    v = jnp.mean((x - m) ** 2, axis=-1, keepdims=True)
    return (x - m) / jnp.sqrt(v + 1e-5)


def _fused_kernel(cx_ref, table1_ref, pos_ref, wq_ref, wk_ref, wv_ref,
                  wo_ref, w1_ref, w2_ref, wout_ref, out_ref):
    cx = cx_ref[...]
    h = jnp.dot(cx, table1_ref[...],
                preferred_element_type=jnp.float32) + pos_ref[...]
    q = jnp.dot(h, wq_ref[...], preferred_element_type=jnp.float32)
    k = jnp.dot(h, wk_ref[...], preferred_element_type=jnp.float32)
    v = jnp.dot(h, wv_ref[...], preferred_element_type=jnp.float32)
    scores = jax.lax.dot_general(
        q, k, (((1,), (1,)), ((), ())),
        preferred_element_type=jnp.float32) * (1.0 / jnp.sqrt(jnp.float32(D)))
    ri = jax.lax.broadcasted_iota(jnp.int32, (R, R), 0) // V
    ci = jax.lax.broadcasted_iota(jnp.int32, (R, R), 1) // V
    scores = jnp.where(ri == ci, scores, -1e30)
    scores = scores - jnp.max(scores, axis=-1, keepdims=True)
    e = jnp.exp(scores)
    attn = e / jnp.sum(e, axis=-1, keepdims=True)
    av = jnp.dot(attn, v, preferred_element_type=jnp.float32)
    h = _layer_norm(h + jnp.dot(av, wo_ref[...],
                                preferred_element_type=jnp.float32))
    ff = jnp.maximum(jnp.dot(h, w1_ref[...],
                             preferred_element_type=jnp.float32), 0.0)
    h = _layer_norm(h + jnp.dot(ff, w2_ref[...],
                                preferred_element_type=jnp.float32))
    pg = jax.lax.broadcasted_iota(jnp.int32, (G, R), 0)
    pr = jax.lax.broadcasted_iota(jnp.int32, (G, R), 1) // V
    pool = jnp.where(pg == pr, jnp.float32(1.0 / V), 0.0)
    pooled = jnp.dot(pool, h, preferred_element_type=jnp.float32)
    out_ref[:, 0, :] = jnp.dot(pooled, wout_ref[...],
                               preferred_element_type=jnp.float32)


def _const_spec(shape):
    return pl.BlockSpec(shape, lambda s: (0,) * len(shape))


@jax.jit
def _run(code_x, table, pos, Wq, Wk, Wv, Wo, W1, W2, Wout):
    table1 = table[1:]
    cx_rows = code_x.reshape(B * V, C)
    pos_rows = jnp.tile(pos, (G, 1))
    out = pl.pallas_call(
        _fused_kernel,
        grid=(B // G,),
        in_specs=[
            pl.BlockSpec((R, C), lambda s: (s, 0)),
            _const_spec((C, D)),
            _const_spec((R, D)),
            _const_spec((D, D)),
            _const_spec((D, D)),
            _const_spec((D, D)),
            _const_spec((D, D)),
            _const_spec((D, DFF)),
            _const_spec((DFF, D)),
            _const_spec((D, C)),
        ],
        out_specs=pl.BlockSpec((G, 1, C), lambda s: (s, 0, 0)),
        out_shape=jax.ShapeDtypeStruct((B, 1, C), jnp.float32),
    )(cx_rows, table1, pos_rows, Wq, Wk, Wv, Wo, W1, W2, Wout)
    return out.reshape(B, C)


def kernel(code_x, divided, neighbors, table, pos, Wq, Wk, Wv, Wo,
           W1, W2, Wout, visit_lens):
    del divided, neighbors, visit_lens
    return _run(code_x, table, pos, Wq, Wk, Wv, Wo, W1, W2, Wout)


# speedup vs baseline: 1.0063x; 1.0063x over previous
import jax
import jax.numpy as jnp
from jax.experimental import pallas as pl

B, V, C = 16, 50, 512
D, DFF = 256, 1024

G = 8
R = G * V


def _layer_norm(x):
    m = jnp.mean(x, axis=-1, keepdims=True)
    v = jnp.mean((x - m) ** 2, axis=-1, keepdims=True)
    return (x - m) / jnp.sqrt(v + 1e-5)


def _fused_kernel(cx_ref, table1_ref, pos_ref, wq_ref, wk_ref, wv_ref,
                  wo_ref, w1_ref, w2_ref, wout_ref, out_ref):
    cx = cx_ref[...]
    h = jnp.dot(cx, table1_ref[...],
                preferred_element_type=jnp.float32) + pos_ref[...]
    q = jnp.dot(h, wq_ref[...], preferred_element_type=jnp.float32)
    k = jnp.dot(h, wk_ref[...], preferred_element_type=jnp.float32)
    v = jnp.dot(h, wv_ref[...], preferred_element_type=jnp.float32)
    scores = jax.lax.dot_general(
        q, k, (((1,), (1,)), ((), ())),
        preferred_element_type=jnp.float32) * (1.0 / jnp.sqrt(jnp.float32(D)))
    ri = jax.lax.broadcasted_iota(jnp.int32, (R, R), 0) // V
    ci = jax.lax.broadcasted_iota(jnp.int32, (R, R), 1) // V
    scores = jnp.where(ri == ci, scores, -1e30)
    scores = scores - jnp.max(scores, axis=-1, keepdims=True)
    e = jnp.exp(scores)
    attn = e / jnp.sum(e, axis=-1, keepdims=True)
    av = jnp.dot(attn, v, preferred_element_type=jnp.float32)
    h = _layer_norm(h + jnp.dot(av, wo_ref[...],
                                preferred_element_type=jnp.float32))
    ff = jnp.maximum(jnp.dot(h, w1_ref[...],
                             preferred_element_type=jnp.float32), 0.0)
    h = _layer_norm(h + jnp.dot(ff, w2_ref[...],
                                preferred_element_type=jnp.float32))
    pg = jax.lax.broadcasted_iota(jnp.int32, (G, R), 0)
    pr = jax.lax.broadcasted_iota(jnp.int32, (G, R), 1) // V
    pool = jnp.where(pg == pr, jnp.float32(1.0 / V), 0.0)
    pooled = jnp.dot(pool, h, preferred_element_type=jnp.float32)
    out_ref[:, 0, :] = jnp.dot(pooled, wout_ref[...],
                               preferred_element_type=jnp.float32)


def _const_spec(shape):
    return pl.BlockSpec(shape, lambda s: (0,) * len(shape))


@jax.jit
def _run(code_x, table, pos, Wq, Wk, Wv, Wo, W1, W2, Wout):
    table1 = table[1:]
    cx_rows = code_x.reshape(B * V, C)
    pos_rows = jnp.tile(pos, (G, 1))
    out = pl.pallas_call(
        _fused_kernel,
        grid=(B // G,),
        in_specs=[
            pl.BlockSpec((R, C), lambda s: (s, 0)),
            _const_spec((C, D)),
            _const_spec((R, D)),
            _const_spec((D, D)),
            _const_spec((D, D)),
            _const_spec((D, D)),
            _const_spec((D, D)),
            _const_spec((D, DFF)),
            _const_spec((DFF, D)),
            _const_spec((D, C)),
        ],
        out_specs=pl.BlockSpec((G, 1, C), lambda s: (s, 0, 0)),
        out_shape=jax.ShapeDtypeStruct((B, 1, C), jnp.float32),
    )(cx_rows, table1, pos_rows, Wq, Wk, Wv, Wo, W1, W2, Wout)
    return out.reshape(B, C)


def kernel(code_x, divided, neighbors, table, pos, Wq, Wk, Wv, Wo,
           W1, W2, Wout, visit_lens):
    del divided, neighbors, visit_lens
    return _run(code_x, table, pos, Wq, Wk, Wv, Wo, W1, W2, Wout)


# speedup vs baseline: 1.1341x; 1.1270x over previous
import jax
import jax.numpy as jnp
from jax.experimental import pallas as pl

B, V, C = 16, 50, 512
D, DFF = 256, 1024

G = 8
R = G * V


def _layer_norm(x):
    m = jnp.mean(x, axis=-1, keepdims=True)
    v = jnp.mean((x - m) ** 2, axis=-1, keepdims=True)
    return (x - m) / jnp.sqrt(v + 1e-5)


def _fused_kernel(cx_ref, table_ref, pos_ref, wq_ref, wk_ref, wv_ref,
                  wo_ref, w1_ref, w2_ref, wout_ref, out_ref):
    cx = cx_ref[...]
    h = jnp.dot(cx, table_ref[pl.ds(1, C), :],
                preferred_element_type=jnp.float32)
    tr = jax.lax.broadcasted_iota(jnp.int32, (R, V), 0) % V
    tv = jax.lax.broadcasted_iota(jnp.int32, (R, V), 1)
    tile_op = jnp.where(tr == tv, jnp.float32(1.0), 0.0)
    h = h + jnp.dot(tile_op, pos_ref[...],
                    preferred_element_type=jnp.float32)
    q = jnp.dot(h, wq_ref[...], preferred_element_type=jnp.float32)
    k = jnp.dot(h, wk_ref[...], preferred_element_type=jnp.float32)
    v = jnp.dot(h, wv_ref[...], preferred_element_type=jnp.float32)
    scores = jax.lax.dot_general(
        q, k, (((1,), (1,)), ((), ())),
        preferred_element_type=jnp.float32) * (1.0 / jnp.sqrt(jnp.float32(D)))
    ri = jax.lax.broadcasted_iota(jnp.int32, (R, R), 0) // V
    ci = jax.lax.broadcasted_iota(jnp.int32, (R, R), 1) // V
    scores = jnp.where(ri == ci, scores, -1e30)
    scores = scores - jnp.max(scores, axis=-1, keepdims=True)
    e = jnp.exp(scores)
    attn = e / jnp.sum(e, axis=-1, keepdims=True)
    av = jnp.dot(attn, v, preferred_element_type=jnp.float32)
    h = _layer_norm(h + jnp.dot(av, wo_ref[...],
                                preferred_element_type=jnp.float32))
    ff = jnp.maximum(jnp.dot(h, w1_ref[...],
                             preferred_element_type=jnp.float32), 0.0)
    h = _layer_norm(h + jnp.dot(ff, w2_ref[...],
                                preferred_element_type=jnp.float32))
    pg = jax.lax.broadcasted_iota(jnp.int32, (G, R), 0)
    pr = jax.lax.broadcasted_iota(jnp.int32, (G, R), 1) // V
    pool = jnp.where(pg == pr, jnp.float32(1.0 / V), 0.0)
    pooled = jnp.dot(pool, h, preferred_element_type=jnp.float32)
    out_ref[:, 0, :] = jnp.dot(pooled, wout_ref[...],
                               preferred_element_type=jnp.float32)


def _const_spec(shape):
    return pl.BlockSpec(shape, lambda s: (0,) * len(shape))


@jax.jit
def _run(code_x, table, pos, Wq, Wk, Wv, Wo, W1, W2, Wout):
    cx_rows = code_x.reshape(B * V, C)
    out = pl.pallas_call(
        _fused_kernel,
        grid=(B // G,),
        in_specs=[
            pl.BlockSpec((R, C), lambda s: (s, 0)),
            _const_spec((C + 1, D)),
            _const_spec((V, D)),
            _const_spec((D, D)),
            _const_spec((D, D)),
            _const_spec((D, D)),
            _const_spec((D, D)),
            _const_spec((D, DFF)),
            _const_spec((DFF, D)),
            _const_spec((D, C)),
        ],
        out_specs=pl.BlockSpec((G, 1, C), lambda s: (s, 0, 0)),
        out_shape=jax.ShapeDtypeStruct((B, 1, C), jnp.float32),
    )(cx_rows, table, pos, Wq, Wk, Wv, Wo, W1, W2, Wout)
    return out.reshape(B, C)


def kernel(code_x, divided, neighbors, table, pos, Wq, Wk, Wv, Wo,
           W1, W2, Wout, visit_lens):
    del divided, neighbors, visit_lens
    return _run(code_x, table, pos, Wq, Wk, Wv, Wo, W1, W2, Wout)
